# baseline (speedup 1.0000x reference)
import jax
import jax.numpy as jnp
from jax.experimental import pallas as pl
from jax.experimental.pallas import tpu as pltpu

_N = 10000
_D = 128
_BM = 200


def _g1_kernel(x_ref, w1_ref, b1_ref, o_ref):
    acc = jnp.dot(x_ref[...], w1_ref[...],
                  preferred_element_type=jnp.float32) + b1_ref[...]
    o_ref[...] = acc.astype(jnp.bfloat16)


def _pass1_kernel(adj_ref, g1_ref, w2_ref, b2_ref, o_ref):
    a = adj_ref[...].astype(jnp.bfloat16)
    h = jnp.dot(a, g1_ref[...], preferred_element_type=jnp.float32)
    h = jnp.maximum(h, 0.0)
    g2 = jnp.dot(h, w2_ref[...],
                 preferred_element_type=jnp.float32) + b2_ref[...]
    o_ref[...] = g2.astype(jnp.bfloat16)


def _pass2_kernel(adj_ref, g2_ref, w3_ref, b3_ref, w4_ref, b4_ref,
                  emb_ref, z_ref):
    a = adj_ref[...].astype(jnp.bfloat16)
    emb = jnp.dot(a, g2_ref[...], preferred_element_type=jnp.float32)
    emb_ref[...] = emb
    t = jnp.dot(emb, w3_ref[...],
                preferred_element_type=jnp.float32) + b3_ref[...]
    t = jnp.maximum(t, 0.0)
    z_ref[...] = jnp.dot(t, w4_ref[...],
                         preferred_element_type=jnp.float32) + b4_ref[...]


def kernel(x, Adj_, W1, b1, W2, b2, W3, b3, W4, b4):
    b1r = b1.reshape(1, _D)
    b2r = b2.reshape(1, _D)
    b3r = b3.reshape(1, _D)
    b4r = b4.reshape(1, _D)
    g1 = pl.pallas_call(
        _g1_kernel,
        out_shape=jax.ShapeDtypeStruct((_N, _D), jnp.bfloat16),
    )(x, W1, b1r)

    n_blocks = _N // _BM
    g2 = pl.pallas_call(
        _pass1_kernel,
        grid=(n_blocks,),
        in_specs=[
            pl.BlockSpec((_BM, _N), lambda i: (i, 0)),
            pl.BlockSpec((_N, _D), lambda i: (0, 0)),
            pl.BlockSpec((_D, _D), lambda i: (0, 0)),
            pl.BlockSpec((1, _D), lambda i: (0, 0)),
        ],
        out_specs=pl.BlockSpec((_BM, _D), lambda i: (i, 0)),
        out_shape=jax.ShapeDtypeStruct((_N, _D), jnp.bfloat16),
    )(Adj_, g1, W2, b2r)

    emb, z = pl.pallas_call(
        _pass2_kernel,
        grid=(n_blocks,),
        in_specs=[
            pl.BlockSpec((_BM, _N), lambda i: (i, 0)),
            pl.BlockSpec((_N, _D), lambda i: (0, 0)),
            pl.BlockSpec((_D, _D), lambda i: (0, 0)),
            pl.BlockSpec((1, _D), lambda i: (0, 0)),
            pl.BlockSpec((_D, _D), lambda i: (0, 0)),
            pl.BlockSpec((1, _D), lambda i: (0, 0)),
        ],
        out_specs=[
            pl.BlockSpec((_BM, _D), lambda i: (i, 0)),
            pl.BlockSpec((_BM, _D), lambda i: (i, 0)),
        ],
        out_shape=[
            jax.ShapeDtypeStruct((_N, _D), jnp.float32),
            jax.ShapeDtypeStruct((_N, _D), jnp.float32),
        ],
    )(Adj_, g2, W3, b3r, W4, b4r)

    return (z, emb)


# speedup vs baseline: 1.0599x; 1.0599x over previous
import jax
import jax.numpy as jnp
from jax.experimental import pallas as pl
from jax.experimental.pallas import tpu as pltpu

_N = 10000
_D = 128
_BM = 400
_NB = _N // _BM


def _fused_kernel(x_ref, adj_ref, w1_ref, b1_ref, w2_ref, b2_ref,
                  w3_ref, b3_ref, w4_ref, b4_ref,
                  emb_ref, z_ref, g1_ref, g2_ref):
    i = pl.program_id(0)

    @pl.when(i == 0)
    def _g1_phase():
        acc = jnp.dot(x_ref[...], w1_ref[...],
                      preferred_element_type=jnp.float32) + b1_ref[...]
        g1_ref[...] = acc.astype(jnp.bfloat16)

    @pl.when((i >= 1) & (i <= _NB))
    def _pass1_phase():
        a = adj_ref[...].astype(jnp.bfloat16)
        h = jnp.dot(a, g1_ref[...], preferred_element_type=jnp.float32)
        h = jnp.maximum(h, 0.0)
        g2 = jnp.dot(h, w2_ref[...],
                     preferred_element_type=jnp.float32) + b2_ref[...]
        g2_ref[pl.ds((i - 1) * _BM, _BM), :] = g2.astype(jnp.bfloat16)

    @pl.when(i > _NB)
    def _pass2_phase():
        a = adj_ref[...].astype(jnp.bfloat16)
        emb = jnp.dot(a, g2_ref[...], preferred_element_type=jnp.float32)
        emb_ref[...] = emb
        t = jnp.dot(emb, w3_ref[...],
                    preferred_element_type=jnp.float32) + b3_ref[...]
        t = jnp.maximum(t, 0.0)
        z_ref[...] = jnp.dot(t, w4_ref[...],
                             preferred_element_type=jnp.float32) + b4_ref[...]


def _adj_map(i):
    return (jnp.where(i <= _NB, jnp.maximum(i - 1, 0), i - 1 - _NB), 0)


def _out_map(i):
    return (jnp.clip(i - 1 - _NB, 0, _NB - 1), 0)


def _const_map(i):
    return (0, 0)


def kernel(x, Adj_, W1, b1, W2, b2, W3, b3, W4, b4):
    full = lambda r, c: pl.BlockSpec((r, c), _const_map)
    emb, z = pl.pallas_call(
        _fused_kernel,
        grid=(1 + 2 * _NB,),
        in_specs=[
            full(_N, _D),
            pl.BlockSpec((_BM, _N), _adj_map),
            full(_D, _D), full(1, _D),
            full(_D, _D), full(1, _D),
            full(_D, _D), full(1, _D),
            full(_D, _D), full(1, _D),
        ],
        out_specs=[
            pl.BlockSpec((_BM, _D), _out_map),
            pl.BlockSpec((_BM, _D), _out_map),
        ],
        out_shape=[
            jax.ShapeDtypeStruct((_N, _D), jnp.float32),
            jax.ShapeDtypeStruct((_N, _D), jnp.float32),
        ],
        scratch_shapes=[
            pltpu.VMEM((_N, _D), jnp.bfloat16),
            pltpu.VMEM((_N, _D), jnp.bfloat16),
        ],
    )(x, Adj_, W1, b1.reshape(1, _D), W2, b2.reshape(1, _D),
      W3, b3.reshape(1, _D), W4, b4.reshape(1, _D))
    return (z, emb)
